# baseline (speedup 1.0000x reference)
import functools

import jax
import jax.numpy as jnp
from jax.experimental import pallas as pl


def _gclstm_step(x_ref, wi_ref, wc_ref, wo_ref, bi_ref, bchi_ref, bc_ref,
                 bchc_ref, bo_ref, bcho_ref, wro_ref, bro_ref, out_ref, hc_ref):
    xb = x_ref[...]
    f32 = jnp.float32
    gi = jnp.dot(xb, wi_ref[...], preferred_element_type=f32) + (bi_ref[...] + bchi_ref[...])
    gc = jnp.dot(xb, wc_ref[...], preferred_element_type=f32) + (bc_ref[...] + bchc_ref[...])
    go = jnp.dot(xb, wo_ref[...], preferred_element_type=f32) + (bo_ref[...] + bcho_ref[...])
    i_gate = 0.5 + 0.5 * jnp.tanh(0.5 * gi)
    t_gate = jnp.tanh(gc)
    o_gate = 0.5 + 0.5 * jnp.tanh(0.5 * go)
    c = i_gate * t_gate
    h = jnp.maximum(o_gate * jnp.tanh(c), 0.0)
    hc_ref[0, :, :] = h
    hc_ref[1, :, :] = c
    out_ref[...] = (
        jnp.dot(h, wro_ref[...], preferred_element_type=f32) + bro_ref[0, 0]
    )


@functools.partial(jax.jit, static_argnames=("block_rows",))
def _run(x, w_i, w_c, w_o, b2_i, bch2_i, b2_c, bch2_c, b2_o, bch2_o, w_ro,
         b_ro, block_rows):
    n, din = x.shape
    demb = w_i.shape[1]
    full = lambda a: pl.BlockSpec(a.shape, lambda i: (0,) * a.ndim)
    out, hc = pl.pallas_call(
        _gclstm_step,
        grid=(pl.cdiv(n, block_rows),),
        in_specs=[
            pl.BlockSpec((block_rows, din), lambda i: (i, 0)),
            full(w_i), full(w_c), full(w_o),
            full(b2_i), full(bch2_i), full(b2_c), full(bch2_c),
            full(b2_o), full(bch2_o),
            full(w_ro), full(b_ro),
        ],
        out_specs=[
            pl.BlockSpec((block_rows, 1), lambda i: (i, 0)),
            pl.BlockSpec((2, block_rows, demb), lambda i: (0, i, 0)),
        ],
        out_shape=[
            jax.ShapeDtypeStruct((n, 1), jnp.float32),
            jax.ShapeDtypeStruct((2, n, demb), jnp.float32),
        ],
    )(x, w_i, w_c, w_o, b2_i, bch2_i, b2_c, bch2_c, b2_o, bch2_o, w_ro, b_ro)
    return out, hc


def kernel(x, edge_index, mask, W_i, W_f, W_c, W_o, b_i, b_f, b_c, b_o,
           Wch_i, Wch_f, Wch_c, Wch_o, bch_i, bch_f, bch_c, bch_o, W_ro, b_ro):
    n = x.shape[0]
    r = lambda b: b.reshape(1, -1)
    block_rows = 1000 if n % 1000 == 0 else 256
    out, hc = _run(x, W_i, W_c, W_o, r(b_i), r(bch_i), r(b_c), r(bch_c),
                   r(b_o), r(bch_o), W_ro, b_ro.reshape(1, 1), block_rows)
    return (out, hc)


# speedup vs baseline: 1.2101x; 1.2101x over previous
import functools

import jax
import jax.numpy as jnp
from jax.experimental import pallas as pl
from jax.experimental.pallas import tpu as pltpu


def _gclstm_step(x_ref, wi_ref, wc_ref, wo_ref, bi_ref, bchi_ref, bc_ref,
                 bchc_ref, bo_ref, bcho_ref, wro_ref, bro_ref, out_ref, hc_ref):
    xb = x_ref[...]
    f32 = jnp.float32
    gi = jnp.dot(xb, wi_ref[...], preferred_element_type=f32) + (bi_ref[...] + bchi_ref[...])
    gc = jnp.dot(xb, wc_ref[...], preferred_element_type=f32) + (bc_ref[...] + bchc_ref[...])
    go = jnp.dot(xb, wo_ref[...], preferred_element_type=f32) + (bo_ref[...] + bcho_ref[...])
    i_gate = 0.5 + 0.5 * jnp.tanh(0.5 * gi)
    t_gate = jnp.tanh(gc)
    o_gate = 0.5 + 0.5 * jnp.tanh(0.5 * go)
    c = i_gate * t_gate
    h = jnp.maximum(o_gate * jnp.tanh(c), 0.0)
    hc_ref[0, :, :] = h
    hc_ref[1, :, :] = c
    out_ref[...] = (
        jnp.dot(h, wro_ref[...], preferred_element_type=f32) + bro_ref[0, 0]
    )


@functools.partial(jax.jit, static_argnames=("block_rows",))
def _run(x, w_i, w_c, w_o, b2_i, bch2_i, b2_c, bch2_c, b2_o, bch2_o, w_ro,
         b_ro, block_rows):
    n, din = x.shape
    demb = w_i.shape[1]
    full = lambda a: pl.BlockSpec(a.shape, lambda i: (0,) * a.ndim)
    out, hc = pl.pallas_call(
        _gclstm_step,
        grid=(pl.cdiv(n, block_rows),),
        compiler_params=pltpu.CompilerParams(
            dimension_semantics=("parallel",)),
        in_specs=[
            pl.BlockSpec((block_rows, din), lambda i: (i, 0)),
            full(w_i), full(w_c), full(w_o),
            full(b2_i), full(bch2_i), full(b2_c), full(bch2_c),
            full(b2_o), full(bch2_o),
            full(w_ro), full(b_ro),
        ],
        out_specs=[
            pl.BlockSpec((block_rows, 1), lambda i: (i, 0)),
            pl.BlockSpec((2, block_rows, demb), lambda i: (0, i, 0)),
        ],
        out_shape=[
            jax.ShapeDtypeStruct((n, 1), jnp.float32),
            jax.ShapeDtypeStruct((2, n, demb), jnp.float32),
        ],
    )(x, w_i, w_c, w_o, b2_i, bch2_i, b2_c, bch2_c, b2_o, bch2_o, w_ro, b_ro)
    return out, hc


def kernel(x, edge_index, mask, W_i, W_f, W_c, W_o, b_i, b_f, b_c, b_o,
           Wch_i, Wch_f, Wch_c, Wch_o, bch_i, bch_f, bch_c, bch_o, W_ro, b_ro):
    n = x.shape[0]
    r = lambda b: b.reshape(1, -1)
    block_rows = 2000 if n % 2000 == 0 else 256
    out, hc = _run(x, W_i, W_c, W_o, r(b_i), r(bch_i), r(b_c), r(bch_c),
                   r(b_o), r(bch_o), W_ro, b_ro.reshape(1, 1), block_rows)
    return (out, hc)
